# Initial kernel scaffold; baseline (speedup 1.0000x reference)
import jax
import jax.numpy as jnp
from jax.experimental import pallas as pl


def kernel(in_tensor, hash_table):
    raise NotImplementedError("write your pallas kernel here")


# speedup vs baseline: 4.7325x; 4.7325x over previous
import functools

import numpy as np
import jax
import jax.numpy as jnp
from jax import lax
from jax.experimental import pallas as pl
from jax.experimental.pallas import tpu as pltpu
from jax.experimental.pallas import tpu_sc as plsc

_NUM_LEVELS = 16
_TABLE_SIZE = 2 ** 19
_FEAT = 2
_GROWTH = np.exp((np.log(1024.0) - np.log(16.0)) / (_NUM_LEVELS - 1))
_SCALINGS = np.floor(16.0 * _GROWTH ** np.arange(_NUM_LEVELS)).astype(np.float32)
_P1 = np.int32(np.uint32(2654435761).astype(np.int32))
_P2 = np.int32(805459861)
_MASK = np.int32(_TABLE_SIZE - 1)

_N = 1048576
_NC = 2
_NS = 16
_NW = _NC * _NS
_PPW = _N // _NW
_C = 128
_VPC = _C // 16
_NCHUNK = _PPW // _C
_NCHUNK_H = _NCHUNK // 2
_PPW_H = _PPW // 2
_LVL_WORDS = _TABLE_SIZE * _FEAT


def _tec_body(xs, ys, zs, tab, scal_in, out,
              sp, scalv,
              cx0, cy0, cz0, cx1, cy1, cz1,
              ofs0, ofs1, idx0, idx1, feat0, feat1,
              outb,
              csem0, csem1, gsem0, gsem1):
    sid = lax.axis_index("s")
    wid = sid * _NC + lax.axis_index("c")
    base0 = wid * _PPW

    pltpu.sync_copy(scal_in.at[pl.ds(0, 32)], scalv)
    iota = lax.iota(jnp.int32, 16)

    cbufs = ((cx0, cy0, cz0), (cx1, cy1, cz1))
    csems = (csem0, csem1)
    ofss = (ofs0, ofs1)
    idxs = (idx0, idx1)
    feats = (feat0, feat1)
    gsems = (gsem0, gsem1)

    def fire_c(c, s):
        b = base0 + c * _C
        pltpu.async_copy(xs.at[pl.ds(b, _C)], cbufs[s][0], csems[s])
        pltpu.async_copy(ys.at[pl.ds(b, _C)], cbufs[s][1], csems[s])
        pltpu.async_copy(zs.at[pl.ds(b, _C)], cbufs[s][2], csems[s])

    def wait_c(s):
        for d, src in enumerate((xs, ys, zs)):
            pltpu.make_async_copy(src.at[pl.ds(0, _C)], cbufs[s][d], csems[s]).wait()

    def fire_g(s):
        for r in range(16):
            pltpu.async_copy(sp.at[idxs[s].at[np.int32(r)]], feats[s].at[np.int32(r)], gsems[s])

    def wait_g(s):
        for r in range(16):
            pltpu.make_async_copy(sp.at[idxs[s].at[np.int32(r)]], feats[s].at[np.int32(r)], gsems[s]).wait()

    def gen(scale, s):
        cx, cy, cz = cbufs[s]
        ofs, idxb = ofss[s], idxs[s]

        def body(v, carry):
            sl = pl.ds(v * 16, 16)
            sx = (cx[sl] * 0.5 + 0.5) * scale
            sy = (cy[sl] * 0.5 + 0.5) * scale
            sz = (cz[sl] * 0.5 + 0.5) * scale
            fxi = sx.astype(jnp.int32)
            fyi = sy.astype(jnp.int32)
            fzi = sz.astype(jnp.int32)
            ofs[0, sl] = sx - fxi.astype(jnp.float32)
            ofs[1, sl] = sy - fyi.astype(jnp.float32)
            ofs[2, sl] = sz - fzi.astype(jnp.float32)
            ax_f = fxi
            ax_c = fxi + 1
            by_f = fyi * _P1
            by_c = by_f + _P1
            cz_f = fzi * _P2
            cz_c = cz_f + _P2
            corners = (
                (ax_c, by_c, cz_c), (ax_c, by_f, cz_c),
                (ax_f, by_f, cz_c), (ax_f, by_c, cz_c),
                (ax_c, by_c, cz_f), (ax_c, by_f, cz_f),
                (ax_f, by_f, cz_f), (ax_f, by_c, cz_f),
            )
            for k, (a, b, c3) in enumerate(corners):
                h2 = ((a ^ b ^ c3) & _MASK) * 2
                idxb[2 * k, sl] = h2
                idxb[2 * k + 1, sl] = h2 + 1
            return carry

        lax.fori_loop(jnp.int32(0), jnp.int32(_VPC), body, jnp.int32(0), unroll=True)

    def interp(c, s):
        ofs, feat = ofss[s], feats[s]

        def body(v, carry):
            sl = pl.ds(v * 16, 16)
            o0 = ofs[0, sl]
            o1 = ofs[1, sl]
            o2 = ofs[2, sl]
            m0 = 1.0 - o0
            m1 = 1.0 - o1
            m2 = 1.0 - o2
            osl = pl.ds(c * _C + v * 16, 16)
            for j in range(2):
                f0 = feat[0 + j, sl]
                f1 = feat[2 + j, sl]
                f2 = feat[4 + j, sl]
                f3 = feat[6 + j, sl]
                f4 = feat[8 + j, sl]
                f5 = feat[10 + j, sl]
                f6 = feat[12 + j, sl]
                f7 = feat[14 + j, sl]
                f03 = f0 * o0 + f3 * m0
                f12 = f1 * o0 + f2 * m0
                f56 = f5 * o0 + f6 * m0
                f47 = f4 * o0 + f7 * m0
                f0312 = f03 * o1 + f12 * m1
                f4756 = f47 * o1 + f56 * m1
                outb[j, osl] = f0312 * o2 + f4756 * m2
            return carry

        lax.fori_loop(jnp.int32(0), jnp.int32(_VPC), body, jnp.int32(0), unroll=True)

    def level_body(l, carry):
        plsc.subcore_barrier()

        @pl.when(sid == jnp.int32(0))
        def _():
            pltpu.sync_copy(tab.at[pl.ds(l * jnp.int32(_LVL_WORDS), _LVL_WORDS)], sp)

        plsc.subcore_barrier()

        scale = scalv[pl.ds(l, 16)][0]
        two_l = l * 2

        for half in range(2):
            cg0 = jnp.int32(half * _NCHUNK_H)

            fire_c(cg0, 0)
            fire_c(cg0 + 1, 1)
            wait_c(0)
            gen(scale, 0)
            fire_c(cg0 + 2, 0)
            fire_g(0)

            def pair_body(pi, carry2, _cg0=cg0):
                c1 = pi * 2 + 1
                c2 = c1 + 1
                wait_c(1)
                gen(scale, 1)

                @pl.when(c1 + 2 < _NCHUNK_H)
                def _():
                    fire_c(_cg0 + c1 + 2, 1)

                fire_g(1)
                wait_g(0)
                interp(c1 - 1, 0)

                @pl.when(c2 < _NCHUNK_H)
                def _():
                    wait_c(0)
                    gen(scale, 0)

                    @pl.when(c2 + 2 < _NCHUNK_H)
                    def _():
                        fire_c(_cg0 + c2 + 2, 0)

                    fire_g(0)

                wait_g(1)
                interp(c1, 1)
                return carry2

            lax.fori_loop(jnp.int32(0), jnp.int32(_NCHUNK_H // 2), pair_body,
                          jnp.int32(0), unroll=False)

            hb = base0 + half * _PPW_H
            pltpu.sync_copy(outb.at[np.int32(0)], out.at[two_l, pl.ds(hb, _PPW_H)])
            pltpu.sync_copy(outb.at[np.int32(1)], out.at[two_l + 1, pl.ds(hb, _PPW_H)])
        return carry

    lax.fori_loop(jnp.int32(0), jnp.int32(_NUM_LEVELS), level_body,
                  jnp.int32(0), unroll=False)


_mesh = plsc.VectorSubcoreMesh(core_axis_name="c", subcore_axis_name="s")

_encode = functools.partial(
    pl.kernel,
    out_type=jax.ShapeDtypeStruct((_NUM_LEVELS * _FEAT, _N), jnp.float32),
    mesh=_mesh,
    scratch_types=[
        pltpu.MemorySpace.VMEM_SHARED((_LVL_WORDS,), jnp.float32),
        pltpu.VMEM((32,), jnp.float32),
        pltpu.VMEM((_C,), jnp.float32), pltpu.VMEM((_C,), jnp.float32),
        pltpu.VMEM((_C,), jnp.float32), pltpu.VMEM((_C,), jnp.float32),
        pltpu.VMEM((_C,), jnp.float32), pltpu.VMEM((_C,), jnp.float32),
        pltpu.VMEM((3, _C), jnp.float32), pltpu.VMEM((3, _C), jnp.float32),
        pltpu.VMEM((16, _C), jnp.int32), pltpu.VMEM((16, _C), jnp.int32),
        pltpu.VMEM((16, _C), jnp.float32), pltpu.VMEM((16, _C), jnp.float32),
        pltpu.VMEM((2, _PPW_H), jnp.float32),
        pltpu.SemaphoreType.DMA, pltpu.SemaphoreType.DMA,
        pltpu.SemaphoreType.DMA, pltpu.SemaphoreType.DMA,
    ],
)(_tec_body)


def kernel(in_tensor, hash_table):
    xt = in_tensor.T.reshape(3, _N)
    xs, ys, zs = xt[0], xt[1], xt[2]
    tab = hash_table.reshape(_NUM_LEVELS * _TABLE_SIZE * _FEAT)
    scal = jnp.asarray(np.concatenate([_SCALINGS, np.zeros(16, np.float32)]), dtype=jnp.float32)
    outT = _encode(xs, ys, zs, tab, scal)
    return outT.T


# speedup vs baseline: 4.7574x; 1.0053x over previous
import functools

import numpy as np
import jax
import jax.numpy as jnp
from jax import lax
from jax.experimental import pallas as pl
from jax.experimental.pallas import tpu as pltpu
from jax.experimental.pallas import tpu_sc as plsc

_NUM_LEVELS = 16
_TABLE_SIZE = 2 ** 19
_FEAT = 2
_GROWTH = np.exp((np.log(1024.0) - np.log(16.0)) / (_NUM_LEVELS - 1))
_SCALINGS = np.floor(16.0 * _GROWTH ** np.arange(_NUM_LEVELS)).astype(np.float32)
_P1 = np.int32(np.uint32(2654435761).astype(np.int32))
_P2 = np.int32(805459861)
_MASK = np.int32(_TABLE_SIZE - 1)

_N = 1048576
_NC = 2
_NS = 16
_NW = _NC * _NS
_PPW = _N // _NW
_C = 128
_VPC = _C // 16
_NCHUNK = _PPW // _C
_NCHUNK_H = _NCHUNK // 2
_PPW_H = _PPW // 2
_LVL_WORDS = _TABLE_SIZE * _FEAT


def _tec_body(xs, ys, zs, tab, scal_in, out,
              sp, scalv,
              cb0, cb1, cb2, cb3,
              ofs0, ofs1, ofs2, ofs3,
              idx0, idx1, idx2, idx3,
              feat0, feat1, feat2, feat3,
              outb,
              csem0, csem1, csem2, csem3,
              gsem0, gsem1, gsem2, gsem3):
    sid = lax.axis_index("s")
    wid = sid * _NC + lax.axis_index("c")
    base0 = wid * _PPW

    pltpu.sync_copy(scal_in.at[pl.ds(0, 32)], scalv)
    iota = lax.iota(jnp.int32, 16)

    cbufs = (cb0, cb1, cb2, cb3)
    csems = (csem0, csem1, csem2, csem3)
    ofss = (ofs0, ofs1, ofs2, ofs3)
    idxs = (idx0, idx1, idx2, idx3)
    feats = (feat0, feat1, feat2, feat3)
    gsems = (gsem0, gsem1, gsem2, gsem3)

    def fire_c(c, s):
        b = base0 + c * _C
        pltpu.async_copy(xs.at[pl.ds(b, _C)], cbufs[s].at[np.int32(0)], csems[s])
        pltpu.async_copy(ys.at[pl.ds(b, _C)], cbufs[s].at[np.int32(1)], csems[s])
        pltpu.async_copy(zs.at[pl.ds(b, _C)], cbufs[s].at[np.int32(2)], csems[s])

    def wait_c(s):
        for d, srcr in enumerate((xs, ys, zs)):
            pltpu.make_async_copy(srcr.at[pl.ds(0, _C)], cbufs[s].at[np.int32(d)], csems[s]).wait()

    def fire_g(s):
        for r in range(16):
            pltpu.async_copy(sp.at[idxs[s].at[np.int32(r)]], feats[s].at[np.int32(r)], gsems[s])

    def wait_g(s):
        for r in range(16):
            pltpu.make_async_copy(sp.at[idxs[s].at[np.int32(r)]], feats[s].at[np.int32(r)], gsems[s]).wait()

    def gen(scale, s):
        cb = cbufs[s]
        ofs, idxb = ofss[s], idxs[s]

        def body(v, carry):
            sl = pl.ds(v * 16, 16)
            sx = (cb[0, sl] * 0.5 + 0.5) * scale
            sy = (cb[1, sl] * 0.5 + 0.5) * scale
            sz = (cb[2, sl] * 0.5 + 0.5) * scale
            fxi = sx.astype(jnp.int32)
            fyi = sy.astype(jnp.int32)
            fzi = sz.astype(jnp.int32)
            ofs[0, sl] = sx - fxi.astype(jnp.float32)
            ofs[1, sl] = sy - fyi.astype(jnp.float32)
            ofs[2, sl] = sz - fzi.astype(jnp.float32)
            ax_f = fxi
            ax_c = fxi + 1
            by_f = fyi * _P1
            by_c = by_f + _P1
            cz_f = fzi * _P2
            cz_c = cz_f + _P2
            corners = (
                (ax_c, by_c, cz_c), (ax_c, by_f, cz_c),
                (ax_f, by_f, cz_c), (ax_f, by_c, cz_c),
                (ax_c, by_c, cz_f), (ax_c, by_f, cz_f),
                (ax_f, by_f, cz_f), (ax_f, by_c, cz_f),
            )
            for k, (a, b, c3) in enumerate(corners):
                h2 = ((a ^ b ^ c3) & _MASK) * 2
                idxb[2 * k, sl] = h2
                idxb[2 * k + 1, sl] = h2 + 1
            return carry

        lax.fori_loop(jnp.int32(0), jnp.int32(_VPC), body, jnp.int32(0), unroll=True)

    def interp(c, s):
        ofs, feat = ofss[s], feats[s]

        def body(v, carry):
            sl = pl.ds(v * 16, 16)
            o0 = ofs[0, sl]
            o1 = ofs[1, sl]
            o2 = ofs[2, sl]
            m0 = 1.0 - o0
            m1 = 1.0 - o1
            m2 = 1.0 - o2
            osl = pl.ds(c * _C + v * 16, 16)
            for j in range(2):
                f0 = feat[0 + j, sl]
                f1 = feat[2 + j, sl]
                f2 = feat[4 + j, sl]
                f3 = feat[6 + j, sl]
                f4 = feat[8 + j, sl]
                f5 = feat[10 + j, sl]
                f6 = feat[12 + j, sl]
                f7 = feat[14 + j, sl]
                f03 = f0 * o0 + f3 * m0
                f12 = f1 * o0 + f2 * m0
                f56 = f5 * o0 + f6 * m0
                f47 = f4 * o0 + f7 * m0
                f0312 = f03 * o1 + f12 * m1
                f4756 = f47 * o1 + f56 * m1
                outb[j, osl] = f0312 * o2 + f4756 * m2
            return carry

        lax.fori_loop(jnp.int32(0), jnp.int32(_VPC), body, jnp.int32(0), unroll=True)

    def level_body(l, carry):
        plsc.subcore_barrier()

        @pl.when(sid == jnp.int32(0))
        def _():
            pltpu.sync_copy(tab.at[pl.ds(l * jnp.int32(_LVL_WORDS), _LVL_WORDS)], sp)

        plsc.subcore_barrier()

        scale = scalv[pl.ds(l, 16)][0]
        two_l = l * 2

        for half in range(2):
            cg0 = jnp.int32(half * _NCHUNK_H)

            for s in range(4):
                fire_c(cg0 + s, s)
            for c in range(3):
                wait_c(c)
                gen(scale, c)
                fire_g(c)
                fire_c(cg0 + c + 4, c)

            def quad_body(qi, carry2, _cg0=cg0):
                cq = qi * 4 + 3
                for u in range(4):
                    s = (3 + u) % 4
                    c = cq + u
                    wait_c(s)
                    gen(scale, s)
                    fire_g(s)

                    @pl.when(c + 4 < _NCHUNK_H)
                    def _(_c=c, _s=s):
                        fire_c(_cg0 + _c + 4, _s)

                    sn = (s + 1) % 4
                    wait_g(sn)
                    interp(c - 3, sn)
                return carry2

            lax.fori_loop(jnp.int32(0), jnp.int32((_NCHUNK_H - 4) // 4), quad_body,
                          jnp.int32(0), unroll=False)

            cl = jnp.int32(_NCHUNK_H - 1)
            wait_c(3)
            gen(scale, 3)
            fire_g(3)
            for u in range(4):
                wait_g(u)
                interp(cl - 3 + u, u)

            hb = base0 + half * _PPW_H
            pltpu.sync_copy(outb.at[np.int32(0)], out.at[two_l, pl.ds(hb, _PPW_H)])
            pltpu.sync_copy(outb.at[np.int32(1)], out.at[two_l + 1, pl.ds(hb, _PPW_H)])
        return carry

    lax.fori_loop(jnp.int32(0), jnp.int32(_NUM_LEVELS), level_body,
                  jnp.int32(0), unroll=False)


_mesh = plsc.VectorSubcoreMesh(core_axis_name="c", subcore_axis_name="s")

_encode = functools.partial(
    pl.kernel,
    out_type=jax.ShapeDtypeStruct((_NUM_LEVELS * _FEAT, _N), jnp.float32),
    mesh=_mesh,
    scratch_types=[
        pltpu.MemorySpace.VMEM_SHARED((_LVL_WORDS,), jnp.float32),
        pltpu.VMEM((32,), jnp.float32),
        pltpu.VMEM((3, _C), jnp.float32), pltpu.VMEM((3, _C), jnp.float32),
        pltpu.VMEM((3, _C), jnp.float32), pltpu.VMEM((3, _C), jnp.float32),
        pltpu.VMEM((3, _C), jnp.float32), pltpu.VMEM((3, _C), jnp.float32),
        pltpu.VMEM((3, _C), jnp.float32), pltpu.VMEM((3, _C), jnp.float32),
        pltpu.VMEM((16, _C), jnp.int32), pltpu.VMEM((16, _C), jnp.int32),
        pltpu.VMEM((16, _C), jnp.int32), pltpu.VMEM((16, _C), jnp.int32),
        pltpu.VMEM((16, _C), jnp.float32), pltpu.VMEM((16, _C), jnp.float32),
        pltpu.VMEM((16, _C), jnp.float32), pltpu.VMEM((16, _C), jnp.float32),
        pltpu.VMEM((2, _PPW_H), jnp.float32),
        pltpu.SemaphoreType.DMA, pltpu.SemaphoreType.DMA,
        pltpu.SemaphoreType.DMA, pltpu.SemaphoreType.DMA,
        pltpu.SemaphoreType.DMA, pltpu.SemaphoreType.DMA,
        pltpu.SemaphoreType.DMA, pltpu.SemaphoreType.DMA,
    ],
)(_tec_body)


def kernel(in_tensor, hash_table):
    xt = in_tensor.T.reshape(3, _N)
    xs, ys, zs = xt[0], xt[1], xt[2]
    tab = hash_table.reshape(_NUM_LEVELS * _TABLE_SIZE * _FEAT)
    scal = jnp.asarray(np.concatenate([_SCALINGS, np.zeros(16, np.float32)]), dtype=jnp.float32)
    outT = _encode(xs, ys, zs, tab, scal)
    return outT.T


# speedup vs baseline: 4.7593x; 1.0004x over previous
import functools

import numpy as np
import jax
import jax.numpy as jnp
from jax import lax
from jax.experimental import pallas as pl
from jax.experimental.pallas import tpu as pltpu
from jax.experimental.pallas import tpu_sc as plsc

_NUM_LEVELS = 16
_TABLE_SIZE = 2 ** 19
_FEAT = 2
_GROWTH = np.exp((np.log(1024.0) - np.log(16.0)) / (_NUM_LEVELS - 1))
_SCALINGS = np.floor(16.0 * _GROWTH ** np.arange(_NUM_LEVELS)).astype(np.float32)
_P1 = np.int32(np.uint32(2654435761).astype(np.int32))
_P2 = np.int32(805459861)
_MASK = np.int32(_TABLE_SIZE - 1)

_N = 1048576
_NC = 2
_NS = 16
_NW = _NC * _NS
_PPW = _N // _NW
_C = 128
_VPC = _C // 16
_NCHUNK = _PPW // _C
_NCHUNK_H = _NCHUNK // 2
_PPW_H = _PPW // 2
_LVL_WORDS = _TABLE_SIZE * _FEAT


def _tec_body(cpk, tab, scal_in, out,
              sp, scalv,
              cb0, cb1, cb2, cb3,
              ofs0, ofs1, ofs2, ofs3,
              idx0, idx1, idx2, idx3,
              feat0, feat1, feat2, feat3,
              outb,
              csem0, csem1, csem2, csem3,
              gsem0, gsem1, gsem2, gsem3):
    sid = lax.axis_index("s")
    wid = sid * _NC + lax.axis_index("c")
    base0 = wid * _PPW

    pltpu.sync_copy(scal_in.at[pl.ds(0, 32)], scalv)

    cbufs = (cb0, cb1, cb2, cb3)
    csems = (csem0, csem1, csem2, csem3)
    ofss = (ofs0, ofs1, ofs2, ofs3)
    idxs = (idx0, idx1, idx2, idx3)
    feats = (feat0, feat1, feat2, feat3)
    gsems = (gsem0, gsem1, gsem2, gsem3)

    def fire_c(c, s):
        b = (base0 + c * _C) * 3
        pltpu.async_copy(cpk.at[pl.ds(b, 3 * _C)], cbufs[s], csems[s])

    def wait_c(s):
        pltpu.make_async_copy(cpk.at[pl.ds(0, 3 * _C)], cbufs[s], csems[s]).wait()

    def fire_g(s):
        pltpu.async_copy(sp.at[idxs[s]], feats[s], gsems[s])

    def wait_g(s):
        pltpu.make_async_copy(sp.at[idxs[s]], feats[s], gsems[s]).wait()

    def gen(scale, s):
        cb = cbufs[s]
        ofs, idxb = ofss[s], idxs[s]

        def body(v, carry):
            sl = pl.ds(v * 16, 16)
            sx = (cb[pl.ds(v * 16, 16)] * 0.5 + 0.5) * scale
            sy = (cb[pl.ds(_C + v * 16, 16)] * 0.5 + 0.5) * scale
            sz = (cb[pl.ds(2 * _C + v * 16, 16)] * 0.5 + 0.5) * scale
            fxi = sx.astype(jnp.int32)
            fyi = sy.astype(jnp.int32)
            fzi = sz.astype(jnp.int32)
            ofs[0, sl] = sx - fxi.astype(jnp.float32)
            ofs[1, sl] = sy - fyi.astype(jnp.float32)
            ofs[2, sl] = sz - fzi.astype(jnp.float32)
            ax_f = fxi
            ax_c = fxi + 1
            by_f = fyi * _P1
            by_c = by_f + _P1
            cz_f = fzi * _P2
            cz_c = cz_f + _P2
            corners = (
                (ax_c, by_c, cz_c), (ax_c, by_f, cz_c),
                (ax_f, by_f, cz_c), (ax_f, by_c, cz_c),
                (ax_c, by_c, cz_f), (ax_c, by_f, cz_f),
                (ax_f, by_f, cz_f), (ax_f, by_c, cz_f),
            )
            for k, (a, b, c3) in enumerate(corners):
                h2 = ((a ^ b ^ c3) & _MASK) * 2
                idxb[pl.ds(2 * k * _C + v * 16, 16)] = h2
                idxb[pl.ds((2 * k + 1) * _C + v * 16, 16)] = h2 + 1
            return carry

        lax.fori_loop(jnp.int32(0), jnp.int32(_VPC), body, jnp.int32(0), unroll=True)

    def interp(c, s):
        ofs, feat = ofss[s], feats[s]

        def body(v, carry):
            sl = pl.ds(v * 16, 16)
            o0 = ofs[0, sl]
            o1 = ofs[1, sl]
            o2 = ofs[2, sl]
            m0 = 1.0 - o0
            m1 = 1.0 - o1
            m2 = 1.0 - o2
            osl = pl.ds(c * _C + v * 16, 16)
            for j in range(2):
                f0 = feat[pl.ds((0 + j) * _C + v * 16, 16)]
                f1 = feat[pl.ds((2 + j) * _C + v * 16, 16)]
                f2 = feat[pl.ds((4 + j) * _C + v * 16, 16)]
                f3 = feat[pl.ds((6 + j) * _C + v * 16, 16)]
                f4 = feat[pl.ds((8 + j) * _C + v * 16, 16)]
                f5 = feat[pl.ds((10 + j) * _C + v * 16, 16)]
                f6 = feat[pl.ds((12 + j) * _C + v * 16, 16)]
                f7 = feat[pl.ds((14 + j) * _C + v * 16, 16)]
                f03 = f0 * o0 + f3 * m0
                f12 = f1 * o0 + f2 * m0
                f56 = f5 * o0 + f6 * m0
                f47 = f4 * o0 + f7 * m0
                f0312 = f03 * o1 + f12 * m1
                f4756 = f47 * o1 + f56 * m1
                outb[j, osl] = f0312 * o2 + f4756 * m2
            return carry

        lax.fori_loop(jnp.int32(0), jnp.int32(_VPC), body, jnp.int32(0), unroll=True)

    def level_body(l, carry):
        plsc.subcore_barrier()

        @pl.when(sid == jnp.int32(0))
        def _():
            pltpu.sync_copy(tab.at[pl.ds(l * jnp.int32(_LVL_WORDS), _LVL_WORDS)], sp)

        plsc.subcore_barrier()

        scale = scalv[pl.ds(l, 16)][0]
        two_l = l * 2

        for half in range(2):
            cg0 = jnp.int32(half * _NCHUNK_H)

            for s in range(4):
                fire_c(cg0 + s, s)
            for c in range(3):
                wait_c(c)
                gen(scale, c)
                fire_g(c)
                fire_c(cg0 + c + 4, c)

            def quad_body(qi, carry2, _cg0=cg0):
                cq = qi * 4 + 3
                for u in range(4):
                    s = (3 + u) % 4
                    c = cq + u
                    wait_c(s)
                    gen(scale, s)
                    fire_g(s)

                    @pl.when(c + 4 < _NCHUNK_H)
                    def _(_c=c, _s=s):
                        fire_c(_cg0 + _c + 4, _s)

                    sn = (s + 1) % 4
                    wait_g(sn)
                    interp(c - 3, sn)
                return carry2

            lax.fori_loop(jnp.int32(0), jnp.int32((_NCHUNK_H - 4) // 4), quad_body,
                          jnp.int32(0), unroll=False)

            cl = jnp.int32(_NCHUNK_H - 1)
            wait_c(3)
            gen(scale, 3)
            fire_g(3)
            for u in range(4):
                wait_g(u)
                interp(cl - 3 + u, u)

            hb = base0 + half * _PPW_H
            pltpu.sync_copy(outb.at[np.int32(0)], out.at[two_l, pl.ds(hb, _PPW_H)])
            pltpu.sync_copy(outb.at[np.int32(1)], out.at[two_l + 1, pl.ds(hb, _PPW_H)])
        return carry

    lax.fori_loop(jnp.int32(0), jnp.int32(_NUM_LEVELS), level_body,
                  jnp.int32(0), unroll=False)


_mesh = plsc.VectorSubcoreMesh(core_axis_name="c", subcore_axis_name="s")

_encode = functools.partial(
    pl.kernel,
    out_type=jax.ShapeDtypeStruct((_NUM_LEVELS * _FEAT, _N), jnp.float32),
    mesh=_mesh,
    scratch_types=[
        pltpu.MemorySpace.VMEM_SHARED((_LVL_WORDS,), jnp.float32),
        pltpu.VMEM((32,), jnp.float32),
        pltpu.VMEM((3 * _C,), jnp.float32), pltpu.VMEM((3 * _C,), jnp.float32),
        pltpu.VMEM((3 * _C,), jnp.float32), pltpu.VMEM((3 * _C,), jnp.float32),
        pltpu.VMEM((3, _C), jnp.float32), pltpu.VMEM((3, _C), jnp.float32),
        pltpu.VMEM((3, _C), jnp.float32), pltpu.VMEM((3, _C), jnp.float32),
        pltpu.VMEM((16 * _C,), jnp.int32), pltpu.VMEM((16 * _C,), jnp.int32),
        pltpu.VMEM((16 * _C,), jnp.int32), pltpu.VMEM((16 * _C,), jnp.int32),
        pltpu.VMEM((16 * _C,), jnp.float32), pltpu.VMEM((16 * _C,), jnp.float32),
        pltpu.VMEM((16 * _C,), jnp.float32), pltpu.VMEM((16 * _C,), jnp.float32),
        pltpu.VMEM((2, _PPW_H), jnp.float32),
        pltpu.SemaphoreType.DMA, pltpu.SemaphoreType.DMA,
        pltpu.SemaphoreType.DMA, pltpu.SemaphoreType.DMA,
        pltpu.SemaphoreType.DMA, pltpu.SemaphoreType.DMA,
        pltpu.SemaphoreType.DMA, pltpu.SemaphoreType.DMA,
    ],
)(_tec_body)


def kernel(in_tensor, hash_table):
    xt = in_tensor.T.reshape(3, _N)
    cpk = xt.reshape(3, _N // _C, _C).transpose(1, 0, 2).reshape(3 * _N)
    tab = hash_table.reshape(_NUM_LEVELS * _TABLE_SIZE * _FEAT)
    scal = jnp.asarray(np.concatenate([_SCALINGS, np.zeros(16, np.float32)]), dtype=jnp.float32)
    outT = _encode(cpk, tab, scal)
    return outT.T
